# baseline (speedup 1.0000x reference)
import functools

import jax
import jax.numpy as jnp
from jax import lax
from jax.experimental import pallas as pl
from jax.experimental.pallas import tpu as pltpu
from jax.experimental.pallas import tpu_sc as plsc

_NC = 2
_NS = 16
_NW = _NC * _NS


def _make_sc_gather(V, D, B, chunk):
    R = B // _NW
    n_chunk = R // chunk
    mesh = plsc.VectorSubcoreMesh(core_axis_name="c", subcore_axis_name="s")

    @functools.partial(
        pl.kernel,
        mesh=mesh,
        compiler_params=pltpu.CompilerParams(use_tc_tiling_on_sc=False),
        out_type=jax.ShapeDtypeStruct((B, D), jnp.float32),
        scratch_types=[
            pltpu.VMEM((n_chunk, chunk), jnp.int32),
            pltpu.VMEM((R, D), jnp.float32),
            pltpu.SemaphoreType.DMA,
        ],
    )
    def k(table_hbm, idx_hbm, out_hbm, idx_v, rows_v, sem):
        wid = lax.axis_index("s") * _NC + lax.axis_index("c")
        pltpu.sync_copy(idx_hbm.at[wid], idx_v)
        copies = [
            pltpu.async_copy(
                table_hbm.at[idx_v.at[j]],
                rows_v.at[pl.ds(j * chunk, chunk)],
                sem,
            )
            for j in range(n_chunk)
        ]
        for cp in copies:
            cp.wait()
        pltpu.sync_copy(rows_v, out_hbm.at[pl.ds(wid * R, R)])

    return k


def _sigmoid(x):
    return 1.0 / (1.0 + jnp.exp(-x))


def _lstm_weights(Wih_ref, Whh_ref, bih_ref, bhh_ref):
    Wcat = jnp.concatenate(
        [Wih_ref[...], Whh_ref[...]], axis=1).T.astype(jnp.bfloat16)
    bias = (bih_ref[...] + bhh_ref[...]).reshape(1, -1)
    return Wcat, bias


def _lstm_scan(seq3_ref, Wcat, bias, h_ref, c_ref, T, B, H):
    h_ref[...] = jnp.zeros((B, H), dtype=jnp.float32)
    c_ref[...] = jnp.zeros((B, H), dtype=jnp.float32)

    NS = 2
    Bs = B // NS

    def step(t, _):
        for s in range(NS):
            rows = pl.ds(s * Bs, Bs)
            xh = jnp.concatenate(
                [seq3_ref[rows, t, :], h_ref[rows, :]], axis=1)
            gates = bias + jnp.dot(
                xh.astype(jnp.bfloat16), Wcat,
                preferred_element_type=jnp.float32
            )
            i = _sigmoid(gates[:, 0 * H:1 * H])
            f = _sigmoid(gates[:, 1 * H:2 * H])
            g = jnp.tanh(gates[:, 2 * H:3 * H])
            o = _sigmoid(gates[:, 3 * H:4 * H])
            c = f * c_ref[rows, :] + i * g
            h_ref[rows, :] = o * jnp.tanh(c)
            c_ref[rows, :] = c
        return 0

    lax.fori_loop(0, T, step, 0, unroll=4)
    return h_ref[...]


def _tc1_body(seq3_ref, conv_ref, Wih_ref, Whh_ref, bih_ref, bhh_ref,
              Wout_ref, bout_ref, W0_ref, W1_ref, emb_ref,
              feat_ref, h_ref, c_ref):
    B, H = h_ref.shape
    A = emb_ref.shape[0]
    L = seq3_ref.shape[1]
    bf = jnp.bfloat16

    Wcat, bias = _lstm_weights(Wih_ref, Whh_ref, bih_ref, bhh_ref)
    h = _lstm_scan(seq3_ref, Wcat, bias, h_ref, c_ref, L - 1, B, H)
    case = jnp.dot(h.astype(bf), Wout_ref[...].T.astype(bf),
                   preferred_element_type=jnp.float32)
    case = case + bout_ref[...].reshape(1, -1)

    cur = conv_ref[:, L - 2:L - 1]
    tgt = conv_ref[:, L - 1:L]
    Gc = (lax.broadcasted_iota(jnp.int32, (B, A), 1) == cur).astype(bf)
    Gt = (lax.broadcasted_iota(jnp.int32, (B, A), 1) == tgt).astype(bf)
    Gd = Gc - Gt

    he1 = seq3_ref[:, L - 2, :]
    te1 = seq3_ref[:, L - 1, :]

    def att(W_ref, feats, diff):
        wf = jnp.dot(W_ref[...], feats.astype(bf),
                     preferred_element_type=jnp.float32)
        d2 = jnp.sum(diff * diff, axis=1, keepdims=True)
        vals = jnp.exp(-jnp.sqrt(d2))
        wfc = jnp.dot(Gc, wf.astype(bf), preferred_element_type=jnp.float32)
        rhs = jnp.concatenate(
            [vals * wfc, jnp.broadcast_to(vals, wfc.shape)], axis=1
        ).astype(bf)
        dn = lax.dot_general(Gt, rhs, (((0,), (0,)), ((), ())),
                             preferred_element_type=jnp.float32)
        delta = dn[:, :wfc.shape[1]]
        norm = dn[:, wfc.shape[1]:wfc.shape[1] + 1]
        return jnp.maximum(wf + delta / (norm + 1e-12), 0.0)

    x1 = att(W0_ref, emb_ref[...], he1 + case - te1)
    diff2 = jnp.dot(Gd, x1.astype(bf),
                    preferred_element_type=jnp.float32) + case
    x2 = att(W1_ref, x1, diff2)
    feat_ref[...] = x2


def _tc2_body(seq3_ref, Wih_ref, Whh_ref, bih_ref, bhh_ref,
              Wout_ref, bout_ref, emb_ref, out_ref, h_ref, c_ref):
    B, H = h_ref.shape
    L = seq3_ref.shape[1]
    bf = jnp.bfloat16

    Wcat, bias = _lstm_weights(Wih_ref, Whh_ref, bih_ref, bhh_ref)
    h = _lstm_scan(seq3_ref, Wcat, bias, h_ref, c_ref, L - 2, B, H)
    pre_rel = jnp.dot(h.astype(bf), Wout_ref[...].T.astype(bf),
                      preferred_element_type=jnp.float32)
    pre_head = seq3_ref[:, L - 1, :]
    pre_emb = pre_head + pre_rel + bout_ref[...].reshape(1, -1)

    emb = emb_ref[...]
    pn = jnp.sum(pre_emb * pre_emb, axis=1, keepdims=True)
    en = jnp.sum(emb * emb, axis=1, keepdims=True)
    lhs = jnp.concatenate(
        [-2.0 * pre_emb, jnp.ones((B, 1), jnp.float32)], axis=1).astype(bf)
    rhsm = jnp.concatenate([emb, en], axis=1).astype(bf)
    d2 = pn + lax.dot_general(lhs, rhsm, (((1,), (1,)), ((), ())),
                              preferred_element_type=jnp.float32)
    out_ref[...] = -jnp.sqrt(jnp.maximum(d2, 0.0))


def kernel(conv_data, emb_table, W0, W1, W_ih, W_hh, b_ih, b_hh, W_out, b_out):
    A, F = emb_table.shape
    B, L = conv_data.shape
    H = W_hh.shape[1]

    conv = conv_data.astype(jnp.int32)
    chunk = 80
    idx3 = conv.reshape(_NW, -1, chunk)

    f32 = jnp.float32
    tc1 = pl.pallas_call(
        _tc1_body,
        out_shape=jax.ShapeDtypeStruct((A, F), f32),
        scratch_shapes=[
            pltpu.VMEM((B, H), f32),
            pltpu.VMEM((B, H), f32),
        ],
    )
    tc2 = pl.pallas_call(
        _tc2_body,
        out_shape=jax.ShapeDtypeStruct((B, A), f32),
        scratch_shapes=[
            pltpu.VMEM((B, H), f32),
            pltpu.VMEM((B, H), f32),
        ],
    )

    gather = _make_sc_gather(A, F, B * L, chunk)
    seq1 = gather(emb_table, idx3).reshape(B, L, F)
    conv_feat = tc1(seq1, conv, W_ih, W_hh, b_ih, b_hh, W_out, b_out,
                    W0.astype(jnp.bfloat16), W1.astype(jnp.bfloat16),
                    emb_table)
    seq2 = gather(conv_feat, idx3).reshape(B, L, F)
    logits = tc2(seq2, W_ih, W_hh, b_ih, b_hh, W_out, b_out, emb_table)
    return logits


# speedup vs baseline: 1.1807x; 1.1807x over previous
import functools

import jax
import jax.numpy as jnp
from jax import lax
from jax.experimental import pallas as pl
from jax.experimental.pallas import tpu as pltpu
from jax.experimental.pallas import tpu_sc as plsc

_NC = 2
_NS = 16
_NW = _NC * _NS


def _make_sc_gather(V, D, B, chunk):
    R = B // _NW
    n_chunk = R // chunk
    mesh = plsc.VectorSubcoreMesh(core_axis_name="c", subcore_axis_name="s")

    @functools.partial(
        pl.kernel,
        mesh=mesh,
        compiler_params=pltpu.CompilerParams(use_tc_tiling_on_sc=False),
        out_type=jax.ShapeDtypeStruct((B, D), jnp.float32),
        scratch_types=[
            pltpu.VMEM((n_chunk, chunk), jnp.int32),
            pltpu.VMEM((R, D), jnp.float32),
            pltpu.SemaphoreType.DMA,
        ],
    )
    def k(table_hbm, idx_hbm, out_hbm, idx_v, rows_v, sem):
        wid = lax.axis_index("s") * _NC + lax.axis_index("c")
        pltpu.sync_copy(idx_hbm.at[wid], idx_v)
        copies = [
            pltpu.async_copy(
                table_hbm.at[idx_v.at[j]],
                rows_v.at[pl.ds(j * chunk, chunk)],
                sem,
            )
            for j in range(n_chunk)
        ]
        for cp in copies:
            cp.wait()
        pltpu.sync_copy(rows_v, out_hbm.at[pl.ds(wid * R, R)])

    return k


def _sigmoid(x):
    return 1.0 / (1.0 + jnp.exp(-x))


def _lstm_weights(Wih_ref, Whh_ref, bih_ref, bhh_ref):
    Wcat = jnp.concatenate(
        [Wih_ref[...], Whh_ref[...]], axis=1).T.astype(jnp.bfloat16)
    bias = (bih_ref[...] + bhh_ref[...]).reshape(1, -1)
    return Wcat, bias


def _lstm_scan(seq3_ref, Wcat, bias, h_ref, c_ref, T, B, H):
    h_ref[...] = jnp.zeros((B, H), dtype=jnp.float32)
    c_ref[...] = jnp.zeros((B, H), dtype=jnp.float32)

    def step(t, carry):
        h, c = carry
        xh = jnp.concatenate([seq3_ref[:, t, :], h], axis=1)
        gates = bias + jnp.dot(
            xh.astype(jnp.bfloat16), Wcat, preferred_element_type=jnp.float32
        )
        i = _sigmoid(gates[:, 0 * H:1 * H])
        f = _sigmoid(gates[:, 1 * H:2 * H])
        g = jnp.tanh(gates[:, 2 * H:3 * H])
        o = _sigmoid(gates[:, 3 * H:4 * H])
        c = f * c + i * g
        h = o * jnp.tanh(c)
        return (h, c)

    zero = jnp.zeros((B, H), dtype=jnp.float32)
    h, _ = lax.fori_loop(0, T, step, (zero, zero), unroll=4)
    return h


def _tc1_body(seq3_ref, conv_ref, Wih_ref, Whh_ref, bih_ref, bhh_ref,
              Wout_ref, bout_ref, W0_ref, W1_ref, emb_ref,
              feat_ref, h_ref, c_ref):
    B, H = h_ref.shape
    A = emb_ref.shape[0]
    L = seq3_ref.shape[1]
    bf = jnp.bfloat16

    Wcat, bias = _lstm_weights(Wih_ref, Whh_ref, bih_ref, bhh_ref)
    h = _lstm_scan(seq3_ref, Wcat, bias, h_ref, c_ref, L - 1, B, H)
    case = jnp.dot(h.astype(bf), Wout_ref[...].T.astype(bf),
                   preferred_element_type=jnp.float32)
    case = case + bout_ref[...].reshape(1, -1)

    cur = conv_ref[:, L - 2:L - 1]
    tgt = conv_ref[:, L - 1:L]
    Gc = (lax.broadcasted_iota(jnp.int32, (B, A), 1) == cur).astype(bf)
    Gt = (lax.broadcasted_iota(jnp.int32, (B, A), 1) == tgt).astype(bf)
    Gd = Gc - Gt

    he1 = seq3_ref[:, L - 2, :]
    te1 = seq3_ref[:, L - 1, :]

    def att(W_ref, feats, diff):
        wf = jnp.dot(W_ref[...], feats.astype(bf),
                     preferred_element_type=jnp.float32)
        d2 = jnp.sum(diff * diff, axis=1, keepdims=True)
        vals = jnp.exp(-jnp.sqrt(d2))
        wfc = jnp.dot(Gc, wf.astype(bf), preferred_element_type=jnp.float32)
        rhs = jnp.concatenate(
            [vals * wfc, jnp.broadcast_to(vals, wfc.shape)], axis=1
        ).astype(bf)
        dn = lax.dot_general(Gt, rhs, (((0,), (0,)), ((), ())),
                             preferred_element_type=jnp.float32)
        delta = dn[:, :wfc.shape[1]]
        norm = dn[:, wfc.shape[1]:wfc.shape[1] + 1]
        return jnp.maximum(wf + delta / (norm + 1e-12), 0.0)

    x1 = att(W0_ref, emb_ref[...], he1 + case - te1)
    diff2 = jnp.dot(Gd, x1.astype(bf),
                    preferred_element_type=jnp.float32) + case
    x2 = att(W1_ref, x1, diff2)
    feat_ref[...] = x2


def _tc2_body(seq3_ref, Wih_ref, Whh_ref, bih_ref, bhh_ref,
              Wout_ref, bout_ref, emb_ref, out_ref, h_ref, c_ref):
    B, H = h_ref.shape
    L = seq3_ref.shape[1]
    bf = jnp.bfloat16

    Wcat, bias = _lstm_weights(Wih_ref, Whh_ref, bih_ref, bhh_ref)
    h = _lstm_scan(seq3_ref, Wcat, bias, h_ref, c_ref, L - 2, B, H)
    pre_rel = jnp.dot(h.astype(bf), Wout_ref[...].T.astype(bf),
                      preferred_element_type=jnp.float32)
    pre_head = seq3_ref[:, L - 1, :]
    pre_emb = pre_head + pre_rel + bout_ref[...].reshape(1, -1)

    emb = emb_ref[...]
    pn = jnp.sum(pre_emb * pre_emb, axis=1, keepdims=True)
    en = jnp.sum(emb * emb, axis=1, keepdims=True)
    lhs = jnp.concatenate(
        [-2.0 * pre_emb, jnp.ones((B, 1), jnp.float32)], axis=1).astype(bf)
    rhsm = jnp.concatenate([emb, en], axis=1).astype(bf)
    d2 = pn + lax.dot_general(lhs, rhsm, (((1,), (1,)), ((), ())),
                              preferred_element_type=jnp.float32)
    out_ref[...] = -jnp.sqrt(jnp.maximum(d2, 0.0))


def kernel(conv_data, emb_table, W0, W1, W_ih, W_hh, b_ih, b_hh, W_out, b_out):
    A, F = emb_table.shape
    B, L = conv_data.shape
    H = W_hh.shape[1]

    conv = conv_data.astype(jnp.int32)
    chunk = 80
    idx3 = conv.reshape(_NW, -1, chunk)

    f32 = jnp.float32
    tc1 = pl.pallas_call(
        _tc1_body,
        out_shape=jax.ShapeDtypeStruct((A, F), f32),
        scratch_shapes=[
            pltpu.VMEM((B, H), f32),
            pltpu.VMEM((B, H), f32),
        ],
    )
    tc2 = pl.pallas_call(
        _tc2_body,
        out_shape=jax.ShapeDtypeStruct((B, A), f32),
        scratch_shapes=[
            pltpu.VMEM((B, H), f32),
            pltpu.VMEM((B, H), f32),
        ],
    )

    gather = _make_sc_gather(A, F, B * L, chunk)
    seq1 = gather(emb_table, idx3).reshape(B, L, F)
    conv_feat = tc1(seq1, conv, W_ih, W_hh, b_ih, b_hh, W_out, b_out,
                    W0.astype(jnp.bfloat16), W1.astype(jnp.bfloat16),
                    emb_table)
    seq2 = gather(conv_feat, idx3).reshape(B, L, F)
    logits = tc2(seq2, W_ih, W_hh, b_ih, b_hh, W_out, b_out, emb_table)
    return logits


# speedup vs baseline: 1.1991x; 1.0157x over previous
import functools

import jax
import jax.numpy as jnp
from jax import lax
from jax.experimental import pallas as pl
from jax.experimental.pallas import tpu as pltpu
from jax.experimental.pallas import tpu_sc as plsc

_NC = 2
_NS = 16
_NW = _NC * _NS


def _make_sc_gather(V, D, B, chunk):
    R = B // _NW
    n_chunk = R // chunk
    mesh = plsc.VectorSubcoreMesh(core_axis_name="c", subcore_axis_name="s")

    @functools.partial(
        pl.kernel,
        mesh=mesh,
        compiler_params=pltpu.CompilerParams(use_tc_tiling_on_sc=False),
        out_type=jax.ShapeDtypeStruct((B, D), jnp.float32),
        scratch_types=[
            pltpu.VMEM((n_chunk, chunk), jnp.int32),
            pltpu.VMEM((R, D), jnp.float32),
            pltpu.SemaphoreType.DMA,
        ],
    )
    def k(table_hbm, idx_hbm, out_hbm, idx_v, rows_v, sem):
        wid = lax.axis_index("s") * _NC + lax.axis_index("c")
        pltpu.sync_copy(idx_hbm.at[wid], idx_v)
        copies = [
            pltpu.async_copy(
                table_hbm.at[idx_v.at[j]],
                rows_v.at[pl.ds(j * chunk, chunk)],
                sem,
            )
            for j in range(n_chunk)
        ]
        for cp in copies:
            cp.wait()
        pltpu.sync_copy(rows_v, out_hbm.at[pl.ds(wid * R, R)])

    return k


def _sigmoid(x):
    return 1.0 / (1.0 + jnp.exp(-x))


def _lstm_weights(Wih_ref, Whh_ref, bih_ref, bhh_ref):
    Wcat = jnp.concatenate(
        [Wih_ref[...], Whh_ref[...]], axis=1).T.astype(jnp.bfloat16)
    bias = (bih_ref[...] + bhh_ref[...]).reshape(1, -1)
    return Wcat, bias


def _lstm_scan(seq3_ref, Wcat, bias, h_ref, c_ref, T, B, H):
    h_ref[...] = jnp.zeros((B, H), dtype=jnp.float32)
    c_ref[...] = jnp.zeros((B, H), dtype=jnp.float32)

    def step(t, carry):
        h, c = carry
        xh = jnp.concatenate([seq3_ref[:, t, :], h], axis=1)
        gates = bias + jnp.dot(
            xh.astype(jnp.bfloat16), Wcat, preferred_element_type=jnp.float32
        )
        i = _sigmoid(gates[:, 0 * H:1 * H])
        f = _sigmoid(gates[:, 1 * H:2 * H])
        g = jnp.tanh(gates[:, 2 * H:3 * H])
        o = _sigmoid(gates[:, 3 * H:4 * H])
        c = f * c + i * g
        h = o * jnp.tanh(c)
        return (h, c)

    zero = jnp.zeros((B, H), dtype=jnp.float32)
    h, _ = lax.fori_loop(0, T, step, (zero, zero), unroll=8)
    return h


def _tc1_body(seq3_ref, conv_ref, Wih_ref, Whh_ref, bih_ref, bhh_ref,
              Wout_ref, bout_ref, W0_ref, W1_ref, emb_ref,
              feat_ref, h_ref, c_ref):
    B, H = h_ref.shape
    A = emb_ref.shape[0]
    L = seq3_ref.shape[1]
    bf = jnp.bfloat16

    Wcat, bias = _lstm_weights(Wih_ref, Whh_ref, bih_ref, bhh_ref)
    h = _lstm_scan(seq3_ref, Wcat, bias, h_ref, c_ref, L - 1, B, H)
    case = jnp.dot(h.astype(bf), Wout_ref[...].T.astype(bf),
                   preferred_element_type=jnp.float32)
    case = case + bout_ref[...].reshape(1, -1)

    cur = conv_ref[:, L - 2:L - 1]
    tgt = conv_ref[:, L - 1:L]
    Gc = (lax.broadcasted_iota(jnp.int32, (B, A), 1) == cur).astype(bf)
    Gt = (lax.broadcasted_iota(jnp.int32, (B, A), 1) == tgt).astype(bf)
    Gd = Gc - Gt

    he1 = seq3_ref[:, L - 2, :]
    te1 = seq3_ref[:, L - 1, :]

    def att(W_ref, feats, diff):
        wf = jnp.dot(W_ref[...], feats.astype(bf),
                     preferred_element_type=jnp.float32)
        d2 = jnp.sum(diff * diff, axis=1, keepdims=True)
        vals = jnp.exp(-jnp.sqrt(d2))
        wfc = jnp.dot(Gc, wf.astype(bf), preferred_element_type=jnp.float32)
        rhs = jnp.concatenate(
            [vals * wfc, jnp.broadcast_to(vals, wfc.shape)], axis=1
        ).astype(bf)
        dn = lax.dot_general(Gt, rhs, (((0,), (0,)), ((), ())),
                             preferred_element_type=jnp.float32)
        delta = dn[:, :wfc.shape[1]]
        norm = dn[:, wfc.shape[1]:wfc.shape[1] + 1]
        return jnp.maximum(wf + delta / (norm + 1e-12), 0.0)

    x1 = att(W0_ref, emb_ref[...], he1 + case - te1)
    diff2 = jnp.dot(Gd, x1.astype(bf),
                    preferred_element_type=jnp.float32) + case
    x2 = att(W1_ref, x1, diff2)
    feat_ref[...] = x2


def _tc2_body(seq3_ref, Wih_ref, Whh_ref, bih_ref, bhh_ref,
              Wout_ref, bout_ref, emb_ref, out_ref, h_ref, c_ref):
    B, H = h_ref.shape
    L = seq3_ref.shape[1]
    bf = jnp.bfloat16

    Wcat, bias = _lstm_weights(Wih_ref, Whh_ref, bih_ref, bhh_ref)
    h = _lstm_scan(seq3_ref, Wcat, bias, h_ref, c_ref, L - 2, B, H)
    pre_rel = jnp.dot(h.astype(bf), Wout_ref[...].T.astype(bf),
                      preferred_element_type=jnp.float32)
    pre_head = seq3_ref[:, L - 1, :]
    pre_emb = pre_head + pre_rel + bout_ref[...].reshape(1, -1)

    emb = emb_ref[...]
    pn = jnp.sum(pre_emb * pre_emb, axis=1, keepdims=True)
    en = jnp.sum(emb * emb, axis=1, keepdims=True)
    lhs = jnp.concatenate(
        [-2.0 * pre_emb, jnp.ones((B, 1), jnp.float32)], axis=1).astype(bf)
    rhsm = jnp.concatenate([emb, en], axis=1).astype(bf)
    d2 = pn + lax.dot_general(lhs, rhsm, (((1,), (1,)), ((), ())),
                              preferred_element_type=jnp.float32)
    out_ref[...] = -jnp.sqrt(jnp.maximum(d2, 0.0))


def kernel(conv_data, emb_table, W0, W1, W_ih, W_hh, b_ih, b_hh, W_out, b_out):
    A, F = emb_table.shape
    B, L = conv_data.shape
    H = W_hh.shape[1]

    conv = conv_data.astype(jnp.int32)
    chunk = 80
    idx3 = conv.reshape(_NW, -1, chunk)

    f32 = jnp.float32
    tc1 = pl.pallas_call(
        _tc1_body,
        out_shape=jax.ShapeDtypeStruct((A, F), f32),
        scratch_shapes=[
            pltpu.VMEM((B, H), f32),
            pltpu.VMEM((B, H), f32),
        ],
    )
    tc2 = pl.pallas_call(
        _tc2_body,
        out_shape=jax.ShapeDtypeStruct((B, A), f32),
        scratch_shapes=[
            pltpu.VMEM((B, H), f32),
            pltpu.VMEM((B, H), f32),
        ],
    )

    gather = _make_sc_gather(A, F, B * L, chunk)
    seq1 = gather(emb_table, idx3).reshape(B, L, F)
    conv_feat = tc1(seq1, conv, W_ih, W_hh, b_ih, b_hh, W_out, b_out,
                    W0.astype(jnp.bfloat16), W1.astype(jnp.bfloat16),
                    emb_table)
    seq2 = gather(conv_feat, idx3).reshape(B, L, F)
    logits = tc2(seq2, W_ih, W_hh, b_ih, b_hh, W_out, b_out, emb_table)
    return logits


# speedup vs baseline: 1.2071x; 1.0066x over previous
import functools

import jax
import jax.numpy as jnp
from jax import lax
from jax.experimental import pallas as pl
from jax.experimental.pallas import tpu as pltpu
from jax.experimental.pallas import tpu_sc as plsc

_NC = 2
_NS = 16
_NW = _NC * _NS


def _make_sc_gather(V, D, B, chunk):
    R = B // _NW
    n_chunk = R // chunk
    mesh = plsc.VectorSubcoreMesh(core_axis_name="c", subcore_axis_name="s")

    @functools.partial(
        pl.kernel,
        mesh=mesh,
        compiler_params=pltpu.CompilerParams(use_tc_tiling_on_sc=False),
        out_type=jax.ShapeDtypeStruct((B, D), jnp.float32),
        scratch_types=[
            pltpu.VMEM((n_chunk, chunk), jnp.int32),
            pltpu.VMEM((R, D), jnp.float32),
            pltpu.SemaphoreType.DMA,
        ],
    )
    def k(table_hbm, idx_hbm, out_hbm, idx_v, rows_v, sem):
        wid = lax.axis_index("s") * _NC + lax.axis_index("c")
        pltpu.sync_copy(idx_hbm.at[wid], idx_v)
        copies = [
            pltpu.async_copy(
                table_hbm.at[idx_v.at[j]],
                rows_v.at[pl.ds(j * chunk, chunk)],
                sem,
            )
            for j in range(n_chunk)
        ]
        for cp in copies:
            cp.wait()
        pltpu.sync_copy(rows_v, out_hbm.at[pl.ds(wid * R, R)])

    return k


def _sigmoid(x):
    return jax.nn.sigmoid(x)


def _lstm_weights(Wih_ref, Whh_ref, bih_ref, bhh_ref):
    Wcat = jnp.concatenate(
        [Wih_ref[...], Whh_ref[...]], axis=1).T.astype(jnp.bfloat16)
    bias = (bih_ref[...] + bhh_ref[...]).reshape(1, -1)
    return Wcat, bias


def _lstm_scan(seq3_ref, Wcat, bias, h_ref, c_ref, T, B, H):
    del h_ref, c_ref

    def step(t, carry):
        h, c = carry
        xh = jnp.concatenate([seq3_ref[:, t, :], h], axis=1)
        gates = bias + jnp.dot(
            xh.astype(jnp.bfloat16), Wcat, preferred_element_type=jnp.float32
        )
        i = _sigmoid(gates[:, 0 * H:1 * H])
        f = _sigmoid(gates[:, 1 * H:2 * H])
        g = jnp.tanh(gates[:, 2 * H:3 * H])
        o = _sigmoid(gates[:, 3 * H:4 * H])
        c = f * c + i * g
        h = o * jnp.tanh(c)
        return (h, c)

    zero = jnp.zeros((B, H), dtype=jnp.float32)
    h, _ = lax.fori_loop(0, T, step, (zero, zero), unroll=8)
    return h


def _tc1_body(seq3_ref, conv_ref, Wih_ref, Whh_ref, bih_ref, bhh_ref,
              Wout_ref, bout_ref, W0_ref, W1_ref, emb_ref,
              feat_ref, h_ref, c_ref):
    B, H = h_ref.shape
    A = emb_ref.shape[0]
    L = seq3_ref.shape[1]
    bf = jnp.bfloat16

    Wcat, bias = _lstm_weights(Wih_ref, Whh_ref, bih_ref, bhh_ref)
    h = _lstm_scan(seq3_ref, Wcat, bias, h_ref, c_ref, L - 1, B, H)
    case = jnp.dot(h.astype(bf), Wout_ref[...].T.astype(bf),
                   preferred_element_type=jnp.float32)
    case = case + bout_ref[...].reshape(1, -1)

    cur = conv_ref[:, L - 2:L - 1]
    tgt = conv_ref[:, L - 1:L]
    Gc = (lax.broadcasted_iota(jnp.int32, (B, A), 1) == cur).astype(bf)
    Gt = (lax.broadcasted_iota(jnp.int32, (B, A), 1) == tgt).astype(bf)
    Gd = Gc - Gt

    he1 = seq3_ref[:, L - 2, :]
    te1 = seq3_ref[:, L - 1, :]

    def att(W_ref, feats, diff):
        wf = jnp.dot(W_ref[...], feats.astype(bf),
                     preferred_element_type=jnp.float32)
        d2 = jnp.sum(diff * diff, axis=1, keepdims=True)
        vals = jnp.exp(-jnp.sqrt(d2))
        wfc = jnp.dot(Gc, wf.astype(bf), preferred_element_type=jnp.float32)
        rhs = jnp.concatenate(
            [vals * wfc, jnp.broadcast_to(vals, wfc.shape)], axis=1
        ).astype(bf)
        dn = lax.dot_general(Gt, rhs, (((0,), (0,)), ((), ())),
                             preferred_element_type=jnp.float32)
        delta = dn[:, :wfc.shape[1]]
        norm = dn[:, wfc.shape[1]:wfc.shape[1] + 1]
        return jnp.maximum(wf + delta / (norm + 1e-12), 0.0)

    x1 = att(W0_ref, emb_ref[...], he1 + case - te1)
    diff2 = jnp.dot(Gd, x1.astype(bf),
                    preferred_element_type=jnp.float32) + case
    x2 = att(W1_ref, x1, diff2)
    feat_ref[...] = x2


def _tc2_body(seq3_ref, Wih_ref, Whh_ref, bih_ref, bhh_ref,
              Wout_ref, bout_ref, emb_ref, out_ref, h_ref, c_ref):
    B, H = h_ref.shape
    L = seq3_ref.shape[1]
    bf = jnp.bfloat16

    Wcat, bias = _lstm_weights(Wih_ref, Whh_ref, bih_ref, bhh_ref)
    h = _lstm_scan(seq3_ref, Wcat, bias, h_ref, c_ref, L - 2, B, H)
    pre_rel = jnp.dot(h.astype(bf), Wout_ref[...].T.astype(bf),
                      preferred_element_type=jnp.float32)
    pre_head = seq3_ref[:, L - 1, :]
    pre_emb = pre_head + pre_rel + bout_ref[...].reshape(1, -1)

    emb = emb_ref[...]
    pn = jnp.sum(pre_emb * pre_emb, axis=1, keepdims=True)
    en = jnp.sum(emb * emb, axis=1, keepdims=True)
    lhs = jnp.concatenate(
        [-2.0 * pre_emb, jnp.ones((B, 1), jnp.float32)], axis=1).astype(bf)
    rhsm = jnp.concatenate([emb, en], axis=1).astype(bf)
    d2 = pn + lax.dot_general(lhs, rhsm, (((1,), (1,)), ((), ())),
                              preferred_element_type=jnp.float32)
    out_ref[...] = -jnp.sqrt(jnp.maximum(d2, 0.0))


def kernel(conv_data, emb_table, W0, W1, W_ih, W_hh, b_ih, b_hh, W_out, b_out):
    A, F = emb_table.shape
    B, L = conv_data.shape
    H = W_hh.shape[1]

    conv = conv_data.astype(jnp.int32)
    chunk = 80
    idx3 = conv.reshape(_NW, -1, chunk)

    f32 = jnp.float32
    tc1 = pl.pallas_call(
        _tc1_body,
        out_shape=jax.ShapeDtypeStruct((A, F), f32),
        scratch_shapes=[
            pltpu.VMEM((B, H), f32),
            pltpu.VMEM((B, H), f32),
        ],
    )
    tc2 = pl.pallas_call(
        _tc2_body,
        out_shape=jax.ShapeDtypeStruct((B, A), f32),
        scratch_shapes=[
            pltpu.VMEM((B, H), f32),
            pltpu.VMEM((B, H), f32),
        ],
    )

    gather = _make_sc_gather(A, F, B * L, chunk)
    seq1 = gather(emb_table, idx3).reshape(B, L, F)
    conv_feat = tc1(seq1, conv, W_ih, W_hh, b_ih, b_hh, W_out, b_out,
                    W0.astype(jnp.bfloat16), W1.astype(jnp.bfloat16),
                    emb_table)
    seq2 = gather(conv_feat, idx3).reshape(B, L, F)
    logits = tc2(seq2, W_ih, W_hh, b_ih, b_hh, W_out, b_out, emb_table)
    return logits


# speedup vs baseline: 1.3646x; 1.1305x over previous
import functools

import jax
import jax.numpy as jnp
from jax import lax
from jax.experimental import pallas as pl
from jax.experimental.pallas import tpu as pltpu
from jax.experimental.pallas import tpu_sc as plsc

_NC = 2
_NS = 16
_NW = _NC * _NS


def _make_sc_gather(V, D, B, chunk):
    R = B // _NW
    n_chunk = R // chunk
    mesh = plsc.VectorSubcoreMesh(core_axis_name="c", subcore_axis_name="s")

    @functools.partial(
        pl.kernel,
        mesh=mesh,
        compiler_params=pltpu.CompilerParams(use_tc_tiling_on_sc=False),
        out_type=jax.ShapeDtypeStruct((B, D), jnp.float32),
        scratch_types=[
            pltpu.VMEM((n_chunk, chunk), jnp.int32),
            pltpu.VMEM((R, D), jnp.float32),
            pltpu.SemaphoreType.DMA,
        ],
    )
    def k(table_hbm, idx_hbm, out_hbm, idx_v, rows_v, sem):
        wid = lax.axis_index("s") * _NC + lax.axis_index("c")
        pltpu.sync_copy(idx_hbm.at[wid], idx_v)
        copies = [
            pltpu.async_copy(
                table_hbm.at[idx_v.at[j]],
                rows_v.at[pl.ds(j * chunk, chunk)],
                sem,
            )
            for j in range(n_chunk)
        ]
        for cp in copies:
            cp.wait()
        pltpu.sync_copy(rows_v, out_hbm.at[pl.ds(wid * R, R)])

    return k


def _sigmoid(x):
    return jax.nn.sigmoid(x)


def _unpack(Xp):
    R2, H2 = Xp.shape
    B, H = 2 * R2, H2 // 2
    i0 = lax.broadcasted_iota(jnp.int32, (B, R2), 0)
    i1 = lax.broadcasted_iota(jnp.int32, (B, R2), 1)
    Re = (i0 == 2 * i1).astype(jnp.float32)
    Ro = (i0 == 2 * i1 + 1).astype(jnp.float32)
    return (jnp.dot(Re, Xp[:, :H], preferred_element_type=jnp.float32)
            + jnp.dot(Ro, Xp[:, H:], preferred_element_type=jnp.float32))


def _lstm_weights(Wih_ref, Whh_ref, bih_ref, bhh_ref):
    bf = jnp.bfloat16
    Wih = Wih_ref[...]
    Whh = Whh_ref[...]
    H = Whh.shape[1]
    Z = jnp.zeros((H, H), jnp.float32)
    cols = []
    for q in range(4):
        wi = Wih[q * H:(q + 1) * H, :].T
        wh = Whh[q * H:(q + 1) * H, :].T
        top = jnp.concatenate(
            [jnp.concatenate([wi, Z], axis=1),
             jnp.concatenate([Z, wi], axis=1)], axis=0)
        bot = jnp.concatenate(
            [jnp.concatenate([wh, Z], axis=1),
             jnp.concatenate([Z, wh], axis=1)], axis=0)
        cols.append(jnp.concatenate([top, bot], axis=0))
    Wblk = jnp.concatenate(cols, axis=1).astype(bf)
    b = (bih_ref[...] + bhh_ref[...]).reshape(1, -1)
    bias2 = jnp.concatenate(
        [jnp.concatenate([b[:, q * H:(q + 1) * H]] * 2, axis=1)
         for q in range(4)], axis=1)
    return Wblk, bias2


def _lstm_scan(seq3_ref, Wblk, bias2, T, B, H):
    B2, H2 = B // 2, 2 * H

    def step(t, carry):
        h, c = carry
        xh = jnp.concatenate([seq3_ref[:, t, :], h], axis=1)
        gates = bias2 + jnp.dot(
            xh.astype(jnp.bfloat16), Wblk, preferred_element_type=jnp.float32
        )
        i = _sigmoid(gates[:, 0 * H2:1 * H2])
        f = _sigmoid(gates[:, 1 * H2:2 * H2])
        g = jnp.tanh(gates[:, 2 * H2:3 * H2])
        o = _sigmoid(gates[:, 3 * H2:4 * H2])
        c = f * c + i * g
        h = o * jnp.tanh(c)
        return (h, c)

    zero = jnp.zeros((B2, H2), dtype=jnp.float32)
    h, _ = lax.fori_loop(0, T, step, (zero, zero), unroll=8)
    return h


def _tc1_body(seq3_ref, conv_ref, Wih_ref, Whh_ref, bih_ref, bhh_ref,
              Wout_ref, bout_ref, W0_ref, W1_ref, emb_ref,
              feat_ref, h_ref, c_ref):
    B, H = h_ref.shape
    del c_ref
    A = emb_ref.shape[0]
    L = seq3_ref.shape[1]
    bf = jnp.bfloat16

    Wblk, bias2 = _lstm_weights(Wih_ref, Whh_ref, bih_ref, bhh_ref)
    h2 = _lstm_scan(seq3_ref, Wblk, bias2, L - 1, B, H)
    WoutT = Wout_ref[...].T
    Z = jnp.zeros_like(WoutT)
    WoutB = jnp.concatenate(
        [jnp.concatenate([WoutT, Z], axis=1),
         jnp.concatenate([Z, WoutT], axis=1)], axis=0).astype(bf)
    case2 = jnp.dot(h2.astype(bf), WoutB, preferred_element_type=jnp.float32)
    case = _unpack(case2) + bout_ref[...].reshape(1, -1)

    cur = conv_ref[:, L - 2:L - 1]
    tgt = conv_ref[:, L - 1:L]
    Gc = (lax.broadcasted_iota(jnp.int32, (B, A), 1) == cur).astype(bf)
    Gt = (lax.broadcasted_iota(jnp.int32, (B, A), 1) == tgt).astype(bf)
    Gd = Gc - Gt

    hd1 = _unpack(seq3_ref[:, L - 2, :] - seq3_ref[:, L - 1, :])

    def att(W_ref, feats, diff):
        wf = jnp.dot(W_ref[...], feats.astype(bf),
                     preferred_element_type=jnp.float32)
        d2 = jnp.sum(diff * diff, axis=1, keepdims=True)
        vals = jnp.exp(-jnp.sqrt(d2))
        wfc = jnp.dot(Gc, wf.astype(bf), preferred_element_type=jnp.float32)
        rhs = jnp.concatenate(
            [vals * wfc, jnp.broadcast_to(vals, wfc.shape)], axis=1
        ).astype(bf)
        dn = lax.dot_general(Gt, rhs, (((0,), (0,)), ((), ())),
                             preferred_element_type=jnp.float32)
        delta = dn[:, :wfc.shape[1]]
        norm = dn[:, wfc.shape[1]:wfc.shape[1] + 1]
        return jnp.maximum(wf + delta / (norm + 1e-12), 0.0)

    x1 = att(W0_ref, emb_ref[...], hd1 + case)
    diff2 = jnp.dot(Gd, x1.astype(bf),
                    preferred_element_type=jnp.float32) + case
    x2 = att(W1_ref, x1, diff2)
    feat_ref[...] = x2


def _tc2_body(seq3_ref, Wih_ref, Whh_ref, bih_ref, bhh_ref,
              Wout_ref, bout_ref, emb_ref, out_ref, h_ref, c_ref):
    B, H = h_ref.shape
    del c_ref
    L = seq3_ref.shape[1]
    bf = jnp.bfloat16

    Wblk, bias2 = _lstm_weights(Wih_ref, Whh_ref, bih_ref, bhh_ref)
    h2 = _lstm_scan(seq3_ref, Wblk, bias2, L - 2, B, H)
    WoutT = Wout_ref[...].T
    Z = jnp.zeros_like(WoutT)
    WoutB = jnp.concatenate(
        [jnp.concatenate([WoutT, Z], axis=1),
         jnp.concatenate([Z, WoutT], axis=1)], axis=0).astype(bf)
    pre_rel2 = jnp.dot(h2.astype(bf), WoutB, preferred_element_type=jnp.float32)
    pre_head2 = seq3_ref[:, L - 1, :]
    pre_emb = _unpack(pre_head2 + pre_rel2) + bout_ref[...].reshape(1, -1)

    emb = emb_ref[...]
    pn = jnp.sum(pre_emb * pre_emb, axis=1, keepdims=True)
    en = jnp.sum(emb * emb, axis=1, keepdims=True)
    lhs = jnp.concatenate(
        [-2.0 * pre_emb, jnp.ones((B, 1), jnp.float32)], axis=1).astype(bf)
    rhsm = jnp.concatenate([emb, en], axis=1).astype(bf)
    d2 = pn + lax.dot_general(lhs, rhsm, (((1,), (1,)), ((), ())),
                              preferred_element_type=jnp.float32)
    out_ref[...] = -jnp.sqrt(jnp.maximum(d2, 0.0))


def kernel(conv_data, emb_table, W0, W1, W_ih, W_hh, b_ih, b_hh, W_out, b_out):
    A, F = emb_table.shape
    B, L = conv_data.shape
    H = W_hh.shape[1]

    conv = conv_data.astype(jnp.int32)
    chunk = 80
    idxp = conv.reshape(B // 2, 2, L).swapaxes(1, 2)
    idx3 = idxp.reshape(_NW, -1, chunk)

    f32 = jnp.float32
    tc1 = pl.pallas_call(
        _tc1_body,
        out_shape=jax.ShapeDtypeStruct((A, F), f32),
        scratch_shapes=[
            pltpu.VMEM((B, H), f32),
            pltpu.VMEM((B, H), f32),
        ],
    )
    tc2 = pl.pallas_call(
        _tc2_body,
        out_shape=jax.ShapeDtypeStruct((B, A), f32),
        scratch_shapes=[
            pltpu.VMEM((B, H), f32),
            pltpu.VMEM((B, H), f32),
        ],
    )

    gather = _make_sc_gather(A, F, B * L, chunk)
    seq1 = gather(emb_table, idx3).reshape(B // 2, L, 2 * F)
    conv_feat = tc1(seq1, conv, W_ih, W_hh, b_ih, b_hh, W_out, b_out,
                    W0.astype(jnp.bfloat16), W1.astype(jnp.bfloat16),
                    emb_table)
    seq2 = gather(conv_feat, idx3).reshape(B // 2, L, 2 * F)
    logits = tc2(seq2, W_ih, W_hh, b_ih, b_hh, W_out, b_out, emb_table)
    return logits
